# baseline (speedup 1.0000x reference)
import functools

import jax
import jax.numpy as jnp
from jax import lax
from jax.experimental import pallas as pl
from jax.experimental.pallas import tpu as pltpu
from jax.experimental.pallas import tpu_sc as plsc


def _build(B, D, NTOP, NU2, NI2):
    info = plsc.get_sparse_core_info()
    NC, NS, L = info.num_cores, info.num_subcores, info.num_lanes
    NW = NC * NS
    BPW = B // NW
    CB = 32
    NCH = BPW // CB
    ROWS = CB * NTOP
    NG = ROWS // 128
    NL = D // L
    D2 = 2 * D

    mesh = plsc.VectorSubcoreMesh(core_axis_name="c", subcore_axis_name="s")

    @functools.partial(
        pl.kernel,
        mesh=mesh,
        out_type=jax.ShapeDtypeStruct((B, D), jnp.float32),
        compiler_params=pltpu.CompilerParams(
            use_tc_tiling_on_sc=False, needs_layout_passes=False),
        scratch_types=[
            pltpu.VMEM((CB,), jnp.int32),
            pltpu.VMEM((CB,), jnp.int32),
            pltpu.VMEM((CB,), jnp.int32),
            pltpu.VMEM((ROWS,), jnp.int32),
            pltpu.VMEM((NG, 128), jnp.int32),
            pltpu.VMEM((ROWS,), jnp.int32),
            pltpu.VMEM((ROWS, D2), jnp.float32),
            pltpu.VMEM((CB, D2), jnp.float32),
            pltpu.VMEM((CB, D), jnp.float32),
            pltpu.SemaphoreType.DMA,
        ],
    )
    def sc_kernel(u_hbm, ue2_hbm, ie2_hbm, idx_hbm, out_hbm,
                  u_c, uh_c, upar_c, idx_c, pidx, pcol, items_v, ue_c,
                  out_c, sem):
        wid = lax.axis_index("s") * NC + lax.axis_index("c")
        base = wid * BPW
        iota16 = lax.iota(jnp.int32, 16)

        def chunk_body(cb, carry):
            off = pl.multiple_of(cb * CB, CB)
            pltpu.sync_copy(u_hbm.at[pl.ds(base + off, CB)], u_c)
            pltpu.sync_copy(
                idx_hbm.at[pl.ds((base + off) * NTOP, ROWS)], idx_c)

            for k in range(CB // L):
                v = u_c[pl.ds(k * L, L)]
                uh_c[pl.ds(k * L, L)] = lax.shift_right_logical(v, 1)
                upar_c[pl.ds(k * L, L)] = (v & 1) * D
            for k in range(ROWS // L):
                v = idx_c[pl.ds(k * L, L)]
                pidx[k // 8, pl.ds((k % 8) * L, L)] = (
                    lax.shift_right_logical(v, 1))
                pcol[pl.ds(k * L, L)] = (v & 1) * D

            pltpu.async_copy(ue2_hbm.at[uh_c], ue_c, sem).wait()
            cps = []
            for g in range(NG):
                cps.append(pltpu.async_copy(
                    ie2_hbm.at[pidx.at[g]],
                    items_v.at[pl.ds(g * 128, 128), :], sem))
            for cp in cps:
                cp.wait()

            def bbody(b, carry2):
                rb = b * NTOP
                acc = [jnp.zeros((L,), jnp.float32) for _ in range(NL)]
                cnt = jnp.float32(0.0)
                for j in range(NTOP):
                    pj = plsc.load_gather(pcol, [iota16 * 0 + (rb + j)])
                    m = pj != 0
                    r = []
                    for c in range(NL):
                        lo = items_v[rb + j, pl.ds(c * L, L)]
                        hi = items_v[rb + j, pl.ds(D + c * L, L)]
                        r.append(jnp.where(m, hi, lo))
                    for c in range(NL):
                        acc[c] = acc[c] + r[c]
                    s = (r[0] + r[1]) + (r[2] + r[3])
                    rs = jnp.sum(s)
                    cnt = cnt + (rs != 0.0).astype(jnp.float32)
                dv = lax.broadcast_in_dim(cnt + 1e-12, (L,), ())
                pu = plsc.load_gather(upar_c, [iota16 * 0 + b])
                mu = pu != 0
                for c in range(NL):
                    ulo = ue_c[b, pl.ds(c * L, L)]
                    uhi = ue_c[b, pl.ds(D + c * L, L)]
                    out_c[b, pl.ds(c * L, L)] = (
                        jnp.where(mu, uhi, ulo) + acc[c] / dv)
                return carry2

            lax.fori_loop(0, CB, bbody, 0)
            pltpu.sync_copy(out_c, out_hbm.at[pl.ds(base + off, CB)])
            return carry

        lax.fori_loop(0, NCH, chunk_body, 0)

    return sc_kernel


@functools.lru_cache(maxsize=None)
def _built(B, D, NTOP, NU2, NI2):
    return _build(B, D, NTOP, NU2, NI2)


def kernel(u, user_emb, item_emb, user_top_index):
    B = u.shape[0]
    NU = user_emb.shape[0] - 1
    NI = item_emb.shape[0] - 1
    D = user_emb.shape[1]
    NTOP = user_top_index.shape[1]
    u = u.astype(jnp.int32)
    ue2 = user_emb[:NU].reshape(NU // 2, 2 * D)
    ie2 = item_emb[:NI].reshape(NI // 2, 2 * D)
    idx = jnp.take(user_top_index.astype(jnp.int32), u, axis=0).reshape(-1)
    return _built(B, D, NTOP, NU // 2, NI // 2)(u, ue2, ie2, idx)
